# Initial kernel scaffold; baseline (speedup 1.0000x reference)
import jax
import jax.numpy as jnp
from jax.experimental import pallas as pl


def kernel(x, edge_index, W1, att_src1, att_dst1, bias1, W2, att_src2, att_dst2, bias2):
    raise NotImplementedError("write your pallas kernel here")


# speedup vs baseline: 11.9836x; 11.9836x over previous
import functools

import jax
import jax.numpy as jnp
from jax import lax
from jax.experimental import pallas as pl
from jax.experimental.pallas import tpu as pltpu
from jax.experimental.pallas import tpu_sc as plsc

N = 10000
E = 320000
F = 128

NC = 2
NS = 16
NW = NC * NS

G = 80
EG = E // G
GPW = E // NW // G
RPT = N // NS

_MESH = plsc.VectorSubcoreMesh(core_axis_name="c", subcore_axis_name="s")


def _pre_body(x_ref, w_ref, bds_ref, bdd_ref, xpT_ref, asrcT_ref, adstT_ref,
              amax_ref, *, heads, bn):
    i = pl.program_id(0)
    xp = jnp.dot(x_ref[...], w_ref[...], preferred_element_type=jnp.float32)
    asr = jnp.dot(xp, bds_ref[...], preferred_element_type=jnp.float32)
    adt = jnp.dot(xp, bdd_ref[...], preferred_element_type=jnp.float32)
    asrcT_ref[...] = asr
    adstT_ref[...] = adt
    xpT_ref[...] = jnp.transpose(xp.reshape(bn, heads, F), (1, 0, 2))
    bmax = jnp.max(asr, axis=0)
    bmax = jnp.where(lax.iota(jnp.int32, 16) < heads, bmax, 60.0)
    row = jnp.broadcast_to(bmax[None, :], (8, 16))

    @pl.when(i == 0)
    def _():
        amax_ref[...] = row

    @pl.when(i > 0)
    def _():
        amax_ref[...] = jnp.maximum(amax_ref[...], row)


def _tc_pre(x, w, bd_src, bd_dst, heads):
    c = x.shape[1]
    d = w.shape[1]
    bn = 400
    grid = (N // bn,)
    return pl.pallas_call(
        functools.partial(_pre_body, heads=heads, bn=bn),
        grid=grid,
        in_specs=[
            pl.BlockSpec((bn, c), lambda i: (i, 0)),
            pl.BlockSpec((c, d), lambda i: (0, 0)),
            pl.BlockSpec((d, 16), lambda i: (0, 0)),
            pl.BlockSpec((d, 16), lambda i: (0, 0)),
        ],
        out_specs=[
            pl.BlockSpec((heads, bn, F), lambda i: (0, i, 0)),
            pl.BlockSpec((bn, 16), lambda i: (i, 0)),
            pl.BlockSpec((bn, 16), lambda i: (i, 0)),
            pl.BlockSpec((8, 16), lambda i: (0, 0)),
        ],
        out_shape=[
            jax.ShapeDtypeStruct((heads, N, F), jnp.float32),
            jax.ShapeDtypeStruct((N, 16), jnp.float32),
            jax.ShapeDtypeStruct((N, 16), jnp.float32),
            jax.ShapeDtypeStruct((8, 16), jnp.float32),
        ],
    )(x, w, bd_src, bd_dst)


def _tables_body(asrc_ref, adst_ref, amax_ref, m_ref, eself_ref):
    asr = asrc_ref[...]
    adt = adst_ref[...]
    m = amax_ref[0:1, :] + adt
    m = jnp.where(m > 0, m, 0.2 * m)
    m_ref[...] = m
    al = asr + adt
    al = jnp.where(al > 0, al, 0.2 * al)
    eself_ref[...] = jnp.exp(al - m)


def _tc_tables(asrcT, adstT, amax):
    return pl.pallas_call(
        _tables_body,
        out_shape=[
            jax.ShapeDtypeStruct((N, 16), jnp.float32),
            jax.ShapeDtypeStruct((N, 16), jnp.float32),
        ],
    )(asrcT, adstT, amax)


def _passa_body(src_h, dst_h, asrc_h, adst_h, m_h, ea_h, spart_h,
                srcs, dsts, abuf, bbuf, mbuf, eabuf, zbuf, s_sh, sem):
    c = lax.axis_index("c")
    s = lax.axis_index("s")
    wid = c * NS + s

    @pl.loop(0, RPT)
    def _(i):
        zbuf[i] = jnp.zeros((16,), jnp.float32)

    pltpu.sync_copy(zbuf, s_sh.at[pl.ds(s * RPT, RPT)])
    plsc.subcore_barrier()

    pltpu.sync_copy(src_h.at[wid], srcs)
    pltpu.sync_copy(dst_h.at[wid], dsts)

    @pl.loop(0, GPW)
    def _(g):
        pltpu.async_copy(asrc_h.at[srcs.at[g]], abuf, sem).wait()
        pltpu.async_copy(adst_h.at[dsts.at[g]], bbuf, sem).wait()
        pltpu.async_copy(m_h.at[dsts.at[g]], mbuf, sem).wait()

        @pl.loop(0, G)
        def _(v):
            al = abuf[v] + bbuf[v]
            al = jnp.where(al > 0, al, 0.2 * al)
            eabuf[v] = jnp.exp(al - mbuf[v])

        pltpu.sync_copy(eabuf, s_sh.at[dsts.at[g]], add=True)
        pltpu.sync_copy(eabuf, ea_h.at[wid, g])

    plsc.subcore_barrier()
    pltpu.sync_copy(s_sh.at[pl.ds(s * RPT, RPT)], spart_h.at[c, s])


def _sc_passa(srcR, dstR, asrcT, adstT, mT):
    return pl.kernel(
        _passa_body,
        out_type=[
            jax.ShapeDtypeStruct((NW, GPW, G, 16), jnp.float32),
            jax.ShapeDtypeStruct((NC, NS, RPT, 16), jnp.float32),
        ],
        mesh=_MESH,
        compiler_params=pltpu.CompilerParams(use_tc_tiling_on_sc=False),
        scratch_types=[
            pltpu.VMEM((GPW, G), jnp.int32),
            pltpu.VMEM((GPW, G), jnp.int32),
            pltpu.VMEM((G, 16), jnp.float32),
            pltpu.VMEM((G, 16), jnp.float32),
            pltpu.VMEM((G, 16), jnp.float32),
            pltpu.VMEM((G, 16), jnp.float32),
            pltpu.VMEM((RPT, 16), jnp.float32),
            pltpu.VMEM_SHARED((N, 16), jnp.float32),
            pltpu.SemaphoreType.DMA,
        ],
    )(srcR, dstR, asrcT, adstT, mT)


def _agg_body(src_h, dst_h, ea_h, xp_h, u_h,
              srcs, dsts, idxb, gbuf, eabuf, zbuf, u_sh, sem, *, heads):
    c = lax.axis_index("c")
    s = lax.axis_index("s")
    wid = c * NS + s

    pltpu.sync_copy(src_h.at[wid], srcs)
    pltpu.sync_copy(dst_h.at[wid], dsts)

    @pl.loop(0, GPW)
    def _(i):
        for k in range(8):
            zbuf[i, pl.ds(k * 16, 16)] = jnp.zeros((16,), jnp.float32)

    for hh in range(heads):
        for k in range(5):
            pltpu.sync_copy(zbuf, u_sh.at[pl.ds(s * RPT + k * GPW, GPW)])
        plsc.subcore_barrier()

        @pl.loop(0, GPW)
        def _(g):
            for k in range(5):
                idxb[0, pl.ds(k * 16, 16)] = (
                    srcs[g, pl.ds(k * 16, 16)] + hh * N)
            pltpu.async_copy(xp_h.at[idxb.at[0]], gbuf, sem).wait()
            pltpu.sync_copy(ea_h.at[wid, g], eabuf)

            @pl.loop(0, G)
            def _(e):
                w = eabuf[e][hh]
                for k in range(8):
                    gbuf[e, pl.ds(k * 16, 16)] = gbuf[e, pl.ds(k * 16, 16)] * w

            pltpu.sync_copy(gbuf, u_sh.at[dsts.at[g]], add=True)

        plsc.subcore_barrier()
        pltpu.sync_copy(u_sh.at[pl.ds(s * RPT, RPT)], u_h.at[c, hh, s])


def _sc_agg(srcR, dstR, ea, xpF, heads):
    return pl.kernel(
        functools.partial(_agg_body, heads=heads),
        out_type=jax.ShapeDtypeStruct((NC, heads, NS, RPT, F), jnp.float32),
        mesh=_MESH,
        compiler_params=pltpu.CompilerParams(use_tc_tiling_on_sc=False),
        scratch_types=[
            pltpu.VMEM((GPW, G), jnp.int32),
            pltpu.VMEM((GPW, G), jnp.int32),
            pltpu.VMEM((1, G), jnp.int32),
            pltpu.VMEM((G, F), jnp.float32),
            pltpu.VMEM((G, 16), jnp.float32),
            pltpu.VMEM((GPW, F), jnp.float32),
            pltpu.VMEM_SHARED((N, F), jnp.float32),
            pltpu.SemaphoreType.DMA,
        ],
    )(srcR, dstR, ea, xpF)


def _fin_body(up_ref, xpT_ref, sp_ref, esel_ref, bias_ref, out_ref, *,
              heads, bn, last):
    u = up_ref[0] + up_ref[1]
    ese = esel_ref[...]
    stot = sp_ref[0] + sp_ref[1] + ese
    eseT = jnp.transpose(ese[:, :heads], (1, 0))[:, :, None]
    sT = jnp.transpose(stot[:, :heads], (1, 0))[:, :, None]
    u = u + eseT * xpT_ref[...]
    out = u / (sT + 1e-16)
    if last:
        out = jnp.mean(out, axis=0) + bias_ref[...]
        out_ref[...] = out
    else:
        out = jnp.transpose(out, (1, 0, 2)).reshape(bn, heads * F)
        out = out + bias_ref[...]
        out_ref[...] = jnp.where(out > 0, out, jnp.exp(out) - 1.0)


def _tc_final(uPart, xpT, sPart, eselfT, bias, heads, last):
    bn = 400
    grid = (N // bn,)
    d = heads * F if not last else F
    return pl.pallas_call(
        functools.partial(_fin_body, heads=heads, bn=bn, last=last),
        grid=grid,
        in_specs=[
            pl.BlockSpec((NC, heads, bn, F), lambda i: (0, 0, i, 0)),
            pl.BlockSpec((heads, bn, F), lambda i: (0, i, 0)),
            pl.BlockSpec((NC, bn, 16), lambda i: (0, i, 0)),
            pl.BlockSpec((bn, 16), lambda i: (i, 0)),
            pl.BlockSpec((1, d), lambda i: (0, 0)),
        ],
        out_specs=pl.BlockSpec((bn, d), lambda i: (i, 0)),
        out_shape=jax.ShapeDtypeStruct((N, d), jnp.float32),
    )(uPart, xpT, sPart, eselfT, bias)


def _block_diag_att(att, heads):
    eye = jnp.eye(16, dtype=jnp.float32)[:heads]
    return (att[:, :, None] * eye[:, None, :]).reshape(heads * F, 16)


def _gat_layer(x, srcR, dstR, w, att_src, att_dst, bias, heads, last):
    bd_src = _block_diag_att(att_src.reshape(heads, F), heads)
    bd_dst = _block_diag_att(att_dst.reshape(heads, F), heads)
    xpT, asrcT, adstT, amax = _tc_pre(x, w, bd_src, bd_dst, heads)
    mT, eselfT = _tc_tables(asrcT, adstT, amax)
    ea, sPart = _sc_passa(srcR, dstR, asrcT, adstT, mT)
    uPart = _sc_agg(srcR, dstR, ea, xpT.reshape(heads * N, F), heads)
    d = F if last else heads * F
    return _tc_final(uPart.reshape(NC, heads, N, F), xpT,
                     sPart.reshape(NC, N, 16), eselfT,
                     bias.reshape(1, d), heads, last)


def kernel(x, edge_index, W1, att_src1, att_dst1, bias1,
           W2, att_src2, att_dst2, bias2):
    srcR = edge_index[0].reshape(NW, GPW, G)
    dstR = edge_index[1].reshape(NW, GPW, G)
    x1 = _gat_layer(x, srcR, dstR, W1, att_src1, att_dst1, bias1, 8, False)
    x2 = _gat_layer(x1, srcR, dstR, W2, att_src2, att_dst2, bias2, 1, True)
    return x2
